# baseline (speedup 1.0000x reference)
import functools

import jax
import jax.numpy as jnp
from jax import lax
from jax.experimental import pallas as pl
from jax.experimental.pallas import tpu as pltpu
from jax.experimental.pallas import tpu_sc as plsc

BATCH = 16384
HIST = 20
EMBED_DIM = 32
NUM_ROWS = BATCH * HIST
MAX_ID = 1000000
NC, NS = 2, 16
NW = NC * NS
B_PER_W = BATCH // NW
ROWS_PER_W = NUM_ROWS // NW
CHUNK = 1024
N_CHUNKS = ROWS_PER_W // CHUNK
LANES = 16
HIST_PAD = 24

VBLK = 256
N_FULL_BLOCKS = MAX_ID // VBLK
V_TAIL = MAX_ID - N_FULL_BLOCKS * VBLK
N_EXTRA = N_FULL_BLOCKS - (N_FULL_BLOCKS // NW) * NW
BLKS_EVEN = N_FULL_BLOCKS // NW


def _stage_idx_body(idxt_hbm, out_hbm, ibuf, obuf, sem):
    wid = lax.axis_index("s") * NC + lax.axis_index("c")
    b0 = wid * B_PER_W

    pltpu.async_copy(idxt_hbm.at[pl.ds(0, 16), pl.ds(b0, B_PER_W)],
                     ibuf.at[pl.ds(0, 16)], sem)
    pltpu.async_copy(idxt_hbm.at[pl.ds(16, 4), pl.ds(b0, B_PER_W)],
                     ibuf.at[pl.ds(16, 4)], sem)
    pltpu.make_async_copy(idxt_hbm.at[pl.ds(0, 16), pl.ds(b0, B_PER_W)],
                          ibuf.at[pl.ds(0, 16)], sem).wait()
    pltpu.make_async_copy(idxt_hbm.at[pl.ds(16, 4), pl.ds(b0, B_PER_W)],
                          ibuf.at[pl.ds(16, 4)], sem).wait()

    iota_h = lax.iota(jnp.int32, LANES) * HIST
    for h in range(HIST):
        def row_step(j, _, h=h):
            r0 = j * LANES
            vec = ibuf[h, pl.ds(r0, LANES)]
            plsc.store_scatter(obuf, [iota_h + (r0 * HIST + h)], vec)
            return _

        lax.fori_loop(0, B_PER_W // LANES, row_step, 0, unroll=4)

    pltpu.async_copy(obuf, out_hbm.at[pl.ds(wid * ROWS_PER_W, ROWS_PER_W)],
                     sem).wait()


_WIN = (LANES - 1) * EMBED_DIM + 8


def _transpose_cols(ib, ob, nv, iotas):
    for c in range(EMBED_DIM):
        q, m = divmod(c, 8)
        for r0 in range(0, nv, LANES):
            vec = ib[c, pl.ds(r0, LANES)]
            plsc.store_scatter(
                ob.at[pl.ds(r0 * EMBED_DIM + 8 * q, _WIN)],
                [iotas[m]], vec)


def _stage_table_body(tabt_hbm, out_hbm, ib0, ib1, ob0, ob1, ibt, obt,
                      sem_i, sem_o):
    wid = lax.axis_index("s") * NC + lax.axis_index("c")
    start_blk = jnp.where(wid < N_EXTRA,
                          wid * (BLKS_EVEN + 1),
                          N_EXTRA * (BLKS_EVEN + 1)
                          + (wid - N_EXTRA) * BLKS_EVEN)
    iota_base = lax.iota(jnp.int32, LANES) * EMBED_DIM
    iota_c = [iota_base + m for m in range(8)]
    last_j = jnp.where(wid < N_EXTRA, BLKS_EVEN, BLKS_EVEN - 1)

    def voff(j):
        return (start_blk + j) * VBLK

    def fire_in(j, ib):
        return pltpu.async_copy(tabt_hbm.at[:, pl.ds(voff(j), VBLK)], ib,
                                sem_i)

    def wait_in(ib):
        pltpu.make_async_copy(tabt_hbm.at[:, pl.ds(0, VBLK)], ib,
                              sem_i).wait()

    def fire_out(j, ob):
        return pltpu.async_copy(
            ob, out_hbm.at[pl.ds(voff(j) * EMBED_DIM, VBLK * EMBED_DIM)],
            sem_o)

    def wait_out(ob):
        pltpu.make_async_copy(
            ob, out_hbm.at[pl.ds(0, VBLK * EMBED_DIM)], sem_o).wait()

    fire_in(0, ib0)
    fire_in(1, ib1)
    wait_in(ib0)
    _transpose_cols(ib0, ob0, VBLK, iota_c)
    fire_out(0, ob0)
    fire_in(2, ib0)
    wait_in(ib1)
    _transpose_cols(ib1, ob1, VBLK, iota_c)
    fire_out(1, ob1)
    fire_in(3, ib1)

    def pair_step(jj, carry):
        j0 = 2 * jj
        j1 = j0 + 1
        wait_out(ob0)
        wait_in(ib0)
        _transpose_cols(ib0, ob0, VBLK, iota_c)
        fire_out(j0, ob0)

        @pl.when(j0 + 2 <= last_j)
        def _fire0():
            fire_in(j0 + 2, ib0)

        wait_out(ob1)
        wait_in(ib1)
        _transpose_cols(ib1, ob1, VBLK, iota_c)
        fire_out(j1, ob1)

        @pl.when(j1 + 2 <= last_j)
        def _fire1():
            fire_in(j1 + 2, ib1)

        return carry

    lax.fori_loop(1, BLKS_EVEN // 2, pair_step, 0)

    wait_out(ob0)

    @pl.when(wid < N_EXTRA)
    def _extra():
        wait_in(ib0)
        _transpose_cols(ib0, ob0, VBLK, iota_c)
        fire_out(BLKS_EVEN, ob0)

    wait_out(ob1)

    @pl.when(wid < N_EXTRA)
    def _extra_drain():
        wait_out(ob0)

    @pl.when(wid == NW - 1)
    def _tail():
        v0 = N_FULL_BLOCKS * VBLK
        pltpu.async_copy(tabt_hbm.at[:, pl.ds(v0, V_TAIL)], ibt,
                         sem_i).wait()
        _transpose_cols(ibt, obt, V_TAIL, iota_c)
        pltpu.async_copy(
            obt, out_hbm.at[pl.ds(v0 * EMBED_DIM, V_TAIL * EMBED_DIM)],
            sem_o).wait()


def _gather_body(idx_hbm, table_hbm, out_hbm, idx_v, rows0, rows1, sem_i,
                 sem_g, sem_o):
    wid = lax.axis_index("s") * NC + lax.axis_index("c")
    base = wid * ROWS_PER_W

    pltpu.async_copy(idx_hbm.at[wid], idx_v, sem_i).wait()

    bufs = (rows0, rows1)
    gathers = [None] * N_CHUNKS
    writes = [None] * N_CHUNKS
    for g in range(N_CHUNKS):
        gathers[g] = pltpu.async_copy(
            table_hbm.at[idx_v.at[pl.ds(g * CHUNK, CHUNK)]], bufs[g % 2],
            sem_g)
        if g >= 1:
            if g >= 2:
                writes[g - 2].wait()
            gathers[g - 1].wait()
            writes[g - 1] = pltpu.async_copy(
                bufs[(g - 1) % 2],
                out_hbm.at[pl.ds(base + (g - 1) * CHUNK, CHUNK)], sem_o)
    gathers[N_CHUNKS - 1].wait()
    writes[N_CHUNKS - 2].wait()
    writes[N_CHUNKS - 1] = pltpu.async_copy(
        bufs[(N_CHUNKS - 1) % 2],
        out_hbm.at[pl.ds(base + (N_CHUNKS - 1) * CHUNK, CHUNK)], sem_o)
    writes[N_CHUNKS - 1].wait()


@functools.partial(jax.jit, static_argnames=())
def kernel(indices, table):
    idx_t = indices.astype(jnp.int32).T
    tab_t = table.T
    mesh = plsc.VectorSubcoreMesh(
        core_axis_name="c", subcore_axis_name="s",
        num_cores=NC, num_subcores=NS,
    )
    stage_idx = pl.kernel(
        _stage_idx_body,
        out_type=jax.ShapeDtypeStruct((NUM_ROWS,), jnp.int32),
        mesh=mesh,
        scratch_types=[
            pltpu.VMEM((HIST_PAD, B_PER_W), jnp.int32),
            pltpu.VMEM((ROWS_PER_W,), jnp.int32),
            pltpu.SemaphoreType.DMA,
        ],
        compiler_params=pltpu.CompilerParams(
            use_tc_tiling_on_sc=True, needs_layout_passes=False),
    )
    stage_tab = pl.kernel(
        _stage_table_body,
        out_type=jax.ShapeDtypeStruct((MAX_ID * EMBED_DIM,), jnp.float32),
        mesh=mesh,
        scratch_types=[
            pltpu.VMEM((EMBED_DIM, VBLK), jnp.float32),
            pltpu.VMEM((EMBED_DIM, VBLK), jnp.float32),
            pltpu.VMEM((VBLK * EMBED_DIM,), jnp.float32),
            pltpu.VMEM((VBLK * EMBED_DIM,), jnp.float32),
            pltpu.VMEM((EMBED_DIM, V_TAIL), jnp.float32),
            pltpu.VMEM((V_TAIL * EMBED_DIM,), jnp.float32),
            pltpu.SemaphoreType.DMA,
            pltpu.SemaphoreType.DMA,
        ],
        compiler_params=pltpu.CompilerParams(
            use_tc_tiling_on_sc=True, needs_layout_passes=False),
    )
    idx_flat = stage_idx(idx_t).reshape(NW, ROWS_PER_W)
    tab_lin = stage_tab(tab_t).reshape(MAX_ID, EMBED_DIM)
    run = pl.kernel(
        _gather_body,
        out_type=jax.ShapeDtypeStruct((NUM_ROWS, EMBED_DIM), jnp.float32),
        mesh=mesh,
        scratch_types=[
            pltpu.VMEM((ROWS_PER_W,), jnp.int32),
            pltpu.VMEM((CHUNK, EMBED_DIM), jnp.float32),
            pltpu.VMEM((CHUNK, EMBED_DIM), jnp.float32),
            pltpu.SemaphoreType.DMA,
            pltpu.SemaphoreType.DMA,
            pltpu.SemaphoreType.DMA,
        ],
        compiler_params=pltpu.CompilerParams(
            use_tc_tiling_on_sc=False, needs_layout_passes=False),
    )
    out = run(idx_flat, tab_lin)
    return out.reshape(BATCH, HIST, EMBED_DIM)


# speedup vs baseline: 1.0110x; 1.0110x over previous
import functools

import jax
import jax.numpy as jnp
from jax import lax
from jax.experimental import pallas as pl
from jax.experimental.pallas import tpu as pltpu
from jax.experimental.pallas import tpu_sc as plsc

BATCH = 16384
HIST = 20
EMBED_DIM = 32
NUM_ROWS = BATCH * HIST
MAX_ID = 1000000
NC, NS = 2, 16
NW = NC * NS
B_PER_W = BATCH // NW
ROWS_PER_W = NUM_ROWS // NW
CHUNK = 1024
N_CHUNKS = ROWS_PER_W // CHUNK
LANES = 16
HIST_PAD = 24

VBLK = 768
N_FULL_BLOCKS = MAX_ID // VBLK
V_TAIL = MAX_ID - N_FULL_BLOCKS * VBLK
N_EXTRA = N_FULL_BLOCKS - (N_FULL_BLOCKS // NW) * NW
BLKS_EVEN = N_FULL_BLOCKS // NW


def _stage_idx_body(idxt_hbm, out_hbm, ibuf, obuf, sem):
    wid = lax.axis_index("s") * NC + lax.axis_index("c")
    b0 = wid * B_PER_W

    pltpu.async_copy(idxt_hbm.at[pl.ds(0, 16), pl.ds(b0, B_PER_W)],
                     ibuf.at[pl.ds(0, 16)], sem)
    pltpu.async_copy(idxt_hbm.at[pl.ds(16, 4), pl.ds(b0, B_PER_W)],
                     ibuf.at[pl.ds(16, 4)], sem)
    pltpu.make_async_copy(idxt_hbm.at[pl.ds(0, 16), pl.ds(b0, B_PER_W)],
                          ibuf.at[pl.ds(0, 16)], sem).wait()
    pltpu.make_async_copy(idxt_hbm.at[pl.ds(16, 4), pl.ds(b0, B_PER_W)],
                          ibuf.at[pl.ds(16, 4)], sem).wait()

    iota_h = lax.iota(jnp.int32, LANES) * HIST
    for h in range(HIST):
        def row_step(j, _, h=h):
            r0 = j * LANES
            vec = ibuf[h, pl.ds(r0, LANES)]
            plsc.store_scatter(obuf, [iota_h + (r0 * HIST + h)], vec)
            return _

        lax.fori_loop(0, B_PER_W // LANES, row_step, 0, unroll=4)

    pltpu.async_copy(obuf, out_hbm.at[pl.ds(wid * ROWS_PER_W, ROWS_PER_W)],
                     sem).wait()


_WIN = (LANES - 1) * EMBED_DIM + 8


def _transpose_cols(ib, ob, nv, iotas):
    def row_chunk(step, carry):
        r0 = step * LANES
        base = r0 * EMBED_DIM
        for c in range(EMBED_DIM):
            q, m = divmod(c, 8)
            vec = ib[c, pl.ds(r0, LANES)]
            plsc.store_scatter(
                ob.at[pl.ds(base + 8 * q, _WIN)], [iotas[m]], vec)
        return carry

    lax.fori_loop(0, nv // LANES, row_chunk, 0)


def _stage_table_body(tabt_hbm, out_hbm, ib0, ib1, ob0, ob1, ibt, obt,
                      sem_i, sem_o):
    wid = lax.axis_index("s") * NC + lax.axis_index("c")
    start_blk = jnp.where(wid < N_EXTRA,
                          wid * (BLKS_EVEN + 1),
                          N_EXTRA * (BLKS_EVEN + 1)
                          + (wid - N_EXTRA) * BLKS_EVEN)
    iota_base = lax.iota(jnp.int32, LANES) * EMBED_DIM
    iota_c = [iota_base + m for m in range(8)]
    last_j = jnp.where(wid < N_EXTRA, BLKS_EVEN, BLKS_EVEN - 1)

    def voff(j):
        return (start_blk + j) * VBLK

    def fire_in(j, ib):
        return pltpu.async_copy(tabt_hbm.at[:, pl.ds(voff(j), VBLK)], ib,
                                sem_i)

    def wait_in(ib):
        pltpu.make_async_copy(tabt_hbm.at[:, pl.ds(0, VBLK)], ib,
                              sem_i).wait()

    def fire_out(j, ob):
        return pltpu.async_copy(
            ob, out_hbm.at[pl.ds(voff(j) * EMBED_DIM, VBLK * EMBED_DIM)],
            sem_o)

    def wait_out(ob):
        pltpu.make_async_copy(
            ob, out_hbm.at[pl.ds(0, VBLK * EMBED_DIM)], sem_o).wait()

    fire_in(0, ib0)
    fire_in(1, ib1)
    wait_in(ib0)
    _transpose_cols(ib0, ob0, VBLK, iota_c)
    fire_out(0, ob0)
    fire_in(2, ib0)
    wait_in(ib1)
    _transpose_cols(ib1, ob1, VBLK, iota_c)
    fire_out(1, ob1)
    fire_in(3, ib1)

    def pair_step(jj, carry):
        j0 = 2 * jj
        j1 = j0 + 1
        wait_out(ob0)
        wait_in(ib0)
        _transpose_cols(ib0, ob0, VBLK, iota_c)
        fire_out(j0, ob0)

        @pl.when(j0 + 2 <= last_j)
        def _fire0():
            fire_in(j0 + 2, ib0)

        wait_out(ob1)
        wait_in(ib1)
        _transpose_cols(ib1, ob1, VBLK, iota_c)
        fire_out(j1, ob1)

        @pl.when(j1 + 2 <= last_j)
        def _fire1():
            fire_in(j1 + 2, ib1)

        return carry

    lax.fori_loop(1, BLKS_EVEN // 2, pair_step, 0)

    wait_out(ob0)

    @pl.when(wid < N_EXTRA)
    def _extra():
        wait_in(ib0)
        _transpose_cols(ib0, ob0, VBLK, iota_c)
        fire_out(BLKS_EVEN, ob0)

    wait_out(ob1)

    @pl.when(wid < N_EXTRA)
    def _extra_drain():
        wait_out(ob0)

    @pl.when(wid == NW - 1)
    def _tail():
        v0 = N_FULL_BLOCKS * VBLK
        pltpu.async_copy(tabt_hbm.at[:, pl.ds(v0, V_TAIL)], ibt,
                         sem_i).wait()
        _transpose_cols(ibt, obt, V_TAIL, iota_c)
        pltpu.async_copy(
            obt, out_hbm.at[pl.ds(v0 * EMBED_DIM, V_TAIL * EMBED_DIM)],
            sem_o).wait()


def _gather_body(idx_hbm, table_hbm, out_hbm, idx_v, rows0, rows1, sem_i,
                 sem_g, sem_o):
    wid = lax.axis_index("s") * NC + lax.axis_index("c")
    base = wid * ROWS_PER_W

    pltpu.async_copy(idx_hbm.at[wid], idx_v, sem_i).wait()

    bufs = (rows0, rows1)
    gathers = [None] * N_CHUNKS
    writes = [None] * N_CHUNKS
    for g in range(N_CHUNKS):
        gathers[g] = pltpu.async_copy(
            table_hbm.at[idx_v.at[pl.ds(g * CHUNK, CHUNK)]], bufs[g % 2],
            sem_g)
        if g >= 1:
            if g >= 2:
                writes[g - 2].wait()
            gathers[g - 1].wait()
            writes[g - 1] = pltpu.async_copy(
                bufs[(g - 1) % 2],
                out_hbm.at[pl.ds(base + (g - 1) * CHUNK, CHUNK)], sem_o)
    gathers[N_CHUNKS - 1].wait()
    writes[N_CHUNKS - 2].wait()
    writes[N_CHUNKS - 1] = pltpu.async_copy(
        bufs[(N_CHUNKS - 1) % 2],
        out_hbm.at[pl.ds(base + (N_CHUNKS - 1) * CHUNK, CHUNK)], sem_o)
    writes[N_CHUNKS - 1].wait()


@functools.partial(jax.jit, static_argnames=())
def kernel(indices, table):
    idx_t = indices.astype(jnp.int32).T
    tab_t = table.T
    mesh = plsc.VectorSubcoreMesh(
        core_axis_name="c", subcore_axis_name="s",
        num_cores=NC, num_subcores=NS,
    )
    stage_idx = pl.kernel(
        _stage_idx_body,
        out_type=jax.ShapeDtypeStruct((NUM_ROWS,), jnp.int32),
        mesh=mesh,
        scratch_types=[
            pltpu.VMEM((HIST_PAD, B_PER_W), jnp.int32),
            pltpu.VMEM((ROWS_PER_W,), jnp.int32),
            pltpu.SemaphoreType.DMA,
        ],
        compiler_params=pltpu.CompilerParams(
            use_tc_tiling_on_sc=True, needs_layout_passes=False),
    )
    stage_tab = pl.kernel(
        _stage_table_body,
        out_type=jax.ShapeDtypeStruct((MAX_ID * EMBED_DIM,), jnp.float32),
        mesh=mesh,
        scratch_types=[
            pltpu.VMEM((EMBED_DIM, VBLK), jnp.float32),
            pltpu.VMEM((EMBED_DIM, VBLK), jnp.float32),
            pltpu.VMEM((VBLK * EMBED_DIM,), jnp.float32),
            pltpu.VMEM((VBLK * EMBED_DIM,), jnp.float32),
            pltpu.VMEM((EMBED_DIM, V_TAIL), jnp.float32),
            pltpu.VMEM((V_TAIL * EMBED_DIM,), jnp.float32),
            pltpu.SemaphoreType.DMA,
            pltpu.SemaphoreType.DMA,
        ],
        compiler_params=pltpu.CompilerParams(
            use_tc_tiling_on_sc=True, needs_layout_passes=False),
    )
    idx_flat = stage_idx(idx_t).reshape(NW, ROWS_PER_W)
    tab_lin = stage_tab(tab_t).reshape(MAX_ID, EMBED_DIM)
    run = pl.kernel(
        _gather_body,
        out_type=jax.ShapeDtypeStruct((NUM_ROWS, EMBED_DIM), jnp.float32),
        mesh=mesh,
        scratch_types=[
            pltpu.VMEM((ROWS_PER_W,), jnp.int32),
            pltpu.VMEM((CHUNK, EMBED_DIM), jnp.float32),
            pltpu.VMEM((CHUNK, EMBED_DIM), jnp.float32),
            pltpu.SemaphoreType.DMA,
            pltpu.SemaphoreType.DMA,
            pltpu.SemaphoreType.DMA,
        ],
        compiler_params=pltpu.CompilerParams(
            use_tc_tiling_on_sc=False, needs_layout_passes=False),
    )
    out = run(idx_flat, tab_lin)
    return out.reshape(BATCH, HIST, EMBED_DIM)


# speedup vs baseline: 1.2056x; 1.1925x over previous
import functools

import jax
import jax.numpy as jnp
from jax import lax
from jax.experimental import pallas as pl
from jax.experimental.pallas import tpu as pltpu
from jax.experimental.pallas import tpu_sc as plsc

BATCH = 16384
HIST = 20
EMBED_DIM = 32
NUM_ROWS = BATCH * HIST
MAX_ID = 1000000
NC, NS = 2, 16
NW = NC * NS
B_PER_W = BATCH // NW
ROWS_PER_W = NUM_ROWS // NW
CHUNK = 1024
N_CHUNKS = ROWS_PER_W // CHUNK
LANES = 16
HIST_PAD = 24


def _stage_idx_body(idxt_hbm, out_hbm, ibuf, obuf, sem):
    wid = lax.axis_index("s") * NC + lax.axis_index("c")
    b0 = wid * B_PER_W

    pltpu.async_copy(idxt_hbm.at[pl.ds(0, 16), pl.ds(b0, B_PER_W)],
                     ibuf.at[pl.ds(0, 16)], sem)
    pltpu.async_copy(idxt_hbm.at[pl.ds(16, 4), pl.ds(b0, B_PER_W)],
                     ibuf.at[pl.ds(16, 4)], sem)
    pltpu.make_async_copy(idxt_hbm.at[pl.ds(0, 16), pl.ds(b0, B_PER_W)],
                          ibuf.at[pl.ds(0, 16)], sem).wait()
    pltpu.make_async_copy(idxt_hbm.at[pl.ds(16, 4), pl.ds(b0, B_PER_W)],
                          ibuf.at[pl.ds(16, 4)], sem).wait()

    iota_h = lax.iota(jnp.int32, LANES) * HIST
    for h in range(HIST):
        def row_step(j, _, h=h):
            r0 = j * LANES
            vec = ibuf[h, pl.ds(r0, LANES)]
            plsc.store_scatter(obuf, [iota_h + (r0 * HIST + h)], vec)
            return _

        lax.fori_loop(0, B_PER_W // LANES, row_step, 0, unroll=4)

    pltpu.async_copy(obuf, out_hbm.at[pl.ds(wid * ROWS_PER_W, ROWS_PER_W)],
                     sem).wait()


def _gather_body(idx_hbm, table_hbm, out_hbm, idx_v, rows0, rows1, sem_i,
                 sem_g, sem_o):
    wid = lax.axis_index("s") * NC + lax.axis_index("c")
    base = wid * ROWS_PER_W

    pltpu.async_copy(idx_hbm.at[wid], idx_v, sem_i).wait()

    bufs = (rows0, rows1)
    gathers = [None] * N_CHUNKS
    writes = [None] * N_CHUNKS
    for g in range(N_CHUNKS):
        gathers[g] = pltpu.async_copy(
            table_hbm.at[idx_v.at[pl.ds(g * CHUNK, CHUNK)]], bufs[g % 2],
            sem_g)
        if g >= 1:
            if g >= 2:
                writes[g - 2].wait()
            gathers[g - 1].wait()
            writes[g - 1] = pltpu.async_copy(
                bufs[(g - 1) % 2],
                out_hbm.at[pl.ds(base + (g - 1) * CHUNK, CHUNK)], sem_o)
    gathers[N_CHUNKS - 1].wait()
    writes[N_CHUNKS - 2].wait()
    writes[N_CHUNKS - 1] = pltpu.async_copy(
        bufs[(N_CHUNKS - 1) % 2],
        out_hbm.at[pl.ds(base + (N_CHUNKS - 1) * CHUNK, CHUNK)], sem_o)
    writes[N_CHUNKS - 1].wait()


@functools.partial(jax.jit, static_argnames=())
def kernel(indices, table):
    idx_t = indices.astype(jnp.int32).T
    mesh = plsc.VectorSubcoreMesh(
        core_axis_name="c", subcore_axis_name="s",
        num_cores=NC, num_subcores=NS,
    )
    stage_idx = pl.kernel(
        _stage_idx_body,
        out_type=jax.ShapeDtypeStruct((NUM_ROWS,), jnp.int32),
        mesh=mesh,
        scratch_types=[
            pltpu.VMEM((HIST_PAD, B_PER_W), jnp.int32),
            pltpu.VMEM((ROWS_PER_W,), jnp.int32),
            pltpu.SemaphoreType.DMA,
        ],
        compiler_params=pltpu.CompilerParams(
            use_tc_tiling_on_sc=True, needs_layout_passes=False),
    )
    idx_flat = stage_idx(idx_t).reshape(NW, ROWS_PER_W)
    run = pl.kernel(
        _gather_body,
        out_type=jax.ShapeDtypeStruct((NUM_ROWS, EMBED_DIM), jnp.float32),
        mesh=mesh,
        scratch_types=[
            pltpu.VMEM((ROWS_PER_W,), jnp.int32),
            pltpu.VMEM((CHUNK, EMBED_DIM), jnp.float32),
            pltpu.VMEM((CHUNK, EMBED_DIM), jnp.float32),
            pltpu.SemaphoreType.DMA,
            pltpu.SemaphoreType.DMA,
            pltpu.SemaphoreType.DMA,
        ],
        compiler_params=pltpu.CompilerParams(
            use_tc_tiling_on_sc=False, needs_layout_passes=False),
    )
    out = run(idx_flat, table)
    return out.reshape(BATCH, HIST, EMBED_DIM)


# speedup vs baseline: 1.2067x; 1.0009x over previous
import functools

import jax
import jax.numpy as jnp
from jax import lax
from jax.experimental import pallas as pl
from jax.experimental.pallas import tpu as pltpu
from jax.experimental.pallas import tpu_sc as plsc

BATCH = 16384
HIST = 20
EMBED_DIM = 32
NUM_ROWS = BATCH * HIST
MAX_ID = 1000000
NC, NS = 2, 16
NW = NC * NS
B_PER_W = BATCH // NW
ROWS_PER_W = NUM_ROWS // NW
CHUNK = 1024
N_CHUNKS = ROWS_PER_W // CHUNK
LANES = 16
HIST_PAD = 24


def _stage_idx_body(idxt_hbm, out_hbm, ibuf, obuf, sem):
    wid = lax.axis_index("s") * NC + lax.axis_index("c")
    b0 = wid * B_PER_W

    pltpu.async_copy(idxt_hbm.at[pl.ds(0, 16), pl.ds(b0, B_PER_W)],
                     ibuf.at[pl.ds(0, 16)], sem)
    pltpu.async_copy(idxt_hbm.at[pl.ds(16, 4), pl.ds(b0, B_PER_W)],
                     ibuf.at[pl.ds(16, 4)], sem)
    pltpu.make_async_copy(idxt_hbm.at[pl.ds(0, 16), pl.ds(b0, B_PER_W)],
                          ibuf.at[pl.ds(0, 16)], sem).wait()
    pltpu.make_async_copy(idxt_hbm.at[pl.ds(16, 4), pl.ds(b0, B_PER_W)],
                          ibuf.at[pl.ds(16, 4)], sem).wait()

    iota_h = lax.iota(jnp.int32, LANES) * HIST
    for h in range(HIST):
        def row_step(j, _, h=h):
            r0 = j * LANES
            vec = ibuf[h, pl.ds(r0, LANES)]
            plsc.store_scatter(obuf, [iota_h + (r0 * HIST + h)], vec)
            return _

        lax.fori_loop(0, B_PER_W // LANES, row_step, 0, unroll=4)

    pltpu.async_copy(obuf, out_hbm.at[pl.ds(wid * ROWS_PER_W, ROWS_PER_W)],
                     sem).wait()


def _gather_body(idx_hbm, table_hbm, out_hbm, idx_v, rows0, rows1, sem_i,
                 sem_g, sem_o):
    wid = lax.axis_index("s") * NC + lax.axis_index("c")
    base = wid * ROWS_PER_W

    pltpu.async_copy(idx_hbm.at[wid], idx_v, sem_i).wait()

    bufs = (rows0, rows1)
    gathers = [None] * N_CHUNKS
    writes = [None] * N_CHUNKS
    for g in range(N_CHUNKS):
        gathers[g] = pltpu.async_copy(
            table_hbm.at[idx_v.at[pl.ds(g * CHUNK, CHUNK)]], bufs[g % 2],
            sem_g)
        if g >= 1:
            if g >= 2:
                writes[g - 2].wait()
            gathers[g - 1].wait()
            writes[g - 1] = pltpu.async_copy(
                bufs[(g - 1) % 2],
                out_hbm.at[pl.ds(base + (g - 1) * CHUNK, CHUNK)], sem_o)
    gathers[N_CHUNKS - 1].wait()
    writes[N_CHUNKS - 2].wait()
    writes[N_CHUNKS - 1] = pltpu.async_copy(
        bufs[(N_CHUNKS - 1) % 2],
        out_hbm.at[pl.ds(base + (N_CHUNKS - 1) * CHUNK, CHUNK)], sem_o)
    writes[N_CHUNKS - 1].wait()


def _kernel_impl(indices, table):
    idx_t = indices.astype(jnp.int32).T
    mesh = plsc.VectorSubcoreMesh(
        core_axis_name="c", subcore_axis_name="s",
        num_cores=NC, num_subcores=NS,
    )
    stage_idx = pl.kernel(
        _stage_idx_body,
        out_type=jax.ShapeDtypeStruct((NUM_ROWS,), jnp.int32),
        mesh=mesh,
        scratch_types=[
            pltpu.VMEM((HIST_PAD, B_PER_W), jnp.int32),
            pltpu.VMEM((ROWS_PER_W,), jnp.int32),
            pltpu.SemaphoreType.DMA,
        ],
        compiler_params=pltpu.CompilerParams(
            use_tc_tiling_on_sc=True, needs_layout_passes=False),
    )
    idx_flat = stage_idx(idx_t).reshape(NW, ROWS_PER_W)
    run = pl.kernel(
        _gather_body,
        out_type=jax.ShapeDtypeStruct((NUM_ROWS, EMBED_DIM), jnp.float32),
        mesh=mesh,
        scratch_types=[
            pltpu.VMEM((ROWS_PER_W,), jnp.int32),
            pltpu.VMEM((CHUNK, EMBED_DIM), jnp.float32),
            pltpu.VMEM((CHUNK, EMBED_DIM), jnp.float32),
            pltpu.SemaphoreType.DMA,
            pltpu.SemaphoreType.DMA,
            pltpu.SemaphoreType.DMA,
        ],
        compiler_params=pltpu.CompilerParams(
            use_tc_tiling_on_sc=False, needs_layout_passes=False),
    )
    out = run(idx_flat, table)
    return out.reshape(BATCH, HIST, EMBED_DIM)


kernel = jax.jit(_kernel_impl)
